# baseline (device time: 35439 ns/iter reference)
import jax
import jax.numpy as jnp
from jax import lax
from jax.experimental import pallas as pl
from jax.experimental.pallas import tpu as pltpu

N_LAYERS = 3


def kernel(x, Win0, Wout0, Win1, Wout1, Win2, Wout2):
    b, d_loc = x.shape
    _, h_loc = Win0.shape

    def body(x_ref, win0_ref, wout0_ref, win1_ref, wout1_ref, win2_ref,
             wout2_ref, out_ref, ysend_ref, xsend_ref, ybuf_ref, xbuf_ref,
             ysend_sem, yrecv_sem, xsend_sem, xrecv_sem):
        my_x = lax.axis_index("x")
        my_y = lax.axis_index("y")
        y_peer = (my_x, 1 - my_y)
        x_peer = (1 - my_x, my_y)

        barrier_sem = pltpu.get_barrier_semaphore()
        for nbr in (y_peer, x_peer):
            pl.semaphore_signal(
                barrier_sem, inc=1,
                device_id=nbr, device_id_type=pl.DeviceIdType.MESH,
            )
        pl.semaphore_wait(barrier_sem, 2)

        wins = (win0_ref, win1_ref, win2_ref)
        wouts = (wout0_ref, wout1_ref, wout2_ref)

        act = x_ref[:, :].astype(jnp.bfloat16)
        for l in range(N_LAYERS):
            p1 = jnp.dot(act, wins[l][:, :].astype(jnp.bfloat16),
                         preferred_element_type=jnp.float32)
            ysend_ref[:, :] = p1.astype(jnp.bfloat16)
            rdma_y = pltpu.make_async_remote_copy(
                src_ref=ysend_ref,
                dst_ref=ybuf_ref.at[l],
                send_sem=ysend_sem.at[l],
                recv_sem=yrecv_sem.at[l],
                device_id=y_peer,
                device_id_type=pl.DeviceIdType.MESH,
            )
            rdma_y.start()
            rdma_y.wait()
            h = jnp.maximum(p1 + ybuf_ref[l].astype(jnp.float32), 0.0)
            h = h.astype(jnp.bfloat16)

            p2 = jnp.dot(h, wouts[l][:, :].astype(jnp.bfloat16),
                         preferred_element_type=jnp.float32)
            xsend_ref[:, :] = p2.astype(jnp.bfloat16)
            rdma_x = pltpu.make_async_remote_copy(
                src_ref=xsend_ref,
                dst_ref=xbuf_ref.at[l],
                send_sem=xsend_sem.at[l],
                recv_sem=xrecv_sem.at[l],
                device_id=x_peer,
                device_id_type=pl.DeviceIdType.MESH,
            )
            rdma_x.start()
            rdma_x.wait()
            acc = p2 + xbuf_ref[l].astype(jnp.float32)
            if l == N_LAYERS - 1:
                out_ref[:, :] = acc
            else:
                act = acc.astype(jnp.bfloat16)

    return pl.pallas_call(
        body,
        out_shape=jax.ShapeDtypeStruct((b, d_loc), jnp.float32),
        in_specs=[pl.BlockSpec(memory_space=pltpu.VMEM)] * 7,
        out_specs=pl.BlockSpec(memory_space=pltpu.VMEM),
        scratch_shapes=[
            pltpu.VMEM((b, h_loc), jnp.bfloat16),
            pltpu.VMEM((b, d_loc), jnp.bfloat16),
            pltpu.VMEM((N_LAYERS, b, h_loc), jnp.bfloat16),
            pltpu.VMEM((N_LAYERS, b, d_loc), jnp.bfloat16),
            pltpu.SemaphoreType.DMA((N_LAYERS,)),
            pltpu.SemaphoreType.DMA((N_LAYERS,)),
            pltpu.SemaphoreType.DMA((N_LAYERS,)),
            pltpu.SemaphoreType.DMA((N_LAYERS,)),
        ],
        compiler_params=pltpu.CompilerParams(collective_id=0),
    )(x, Win0, Wout0, Win1, Wout1, Win2, Wout2)


# device time: 30181 ns/iter; 1.1742x vs baseline; 1.1742x over previous
import jax
import jax.numpy as jnp
from jax import lax
from jax.experimental import pallas as pl
from jax.experimental.pallas import tpu as pltpu

N_LAYERS = 3
NC = 2


def kernel(x, Win0, Wout0, Win1, Wout1, Win2, Wout2):
    b, d_loc = x.shape
    d_in, h_loc = Win0.shape
    bc = b // NC

    def rows(c):
        return pl.ds(c * bc, bc)

    def body(x_ref, win0_ref, wout0_ref, win1_ref, wout1_ref, win2_ref,
             wout2_ref, out_ref, winb_ref, woutb_ref, ysend_ref, xsend_ref,
             ybuf_ref, xbuf_ref, ysend_sem, yrecv_sem, xsend_sem, xrecv_sem):
        my_x = lax.axis_index("x")
        my_y = lax.axis_index("y")
        y_peer = (my_x, 1 - my_y)
        x_peer = (1 - my_x, my_y)

        barrier_sem = pltpu.get_barrier_semaphore()
        for nbr in (y_peer, x_peer):
            pl.semaphore_signal(
                barrier_sem, inc=1,
                device_id=nbr, device_id_type=pl.DeviceIdType.MESH,
            )

        wins = (win0_ref, win1_ref, win2_ref)
        wouts = (wout0_ref, wout1_ref, wout2_ref)
        for l in range(N_LAYERS):
            winb_ref[l, :, :] = wins[l][:, :].astype(jnp.bfloat16)
            woutb_ref[l, :, :] = wouts[l][:, :].astype(jnp.bfloat16)

        pl.semaphore_wait(barrier_sem, 2)

        def y_rdma(l, c):
            return pltpu.make_async_remote_copy(
                src_ref=ysend_ref.at[l, rows(c)],
                dst_ref=ybuf_ref.at[l, rows(c)],
                send_sem=ysend_sem.at[l, c],
                recv_sem=yrecv_sem.at[l, c],
                device_id=y_peer,
                device_id_type=pl.DeviceIdType.MESH,
            )

        def x_rdma(l, c):
            return pltpu.make_async_remote_copy(
                src_ref=xsend_ref.at[l, rows(c)],
                dst_ref=xbuf_ref.at[l, rows(c)],
                send_sem=xsend_sem.at[l, c],
                recv_sem=xrecv_sem.at[l, c],
                device_id=x_peer,
                device_id_type=pl.DeviceIdType.MESH,
            )

        act = x_ref[:, :].astype(jnp.bfloat16)
        for c in range(NC):
            p1 = jnp.dot(act[c * bc:(c + 1) * bc, :], winb_ref[0],
                         preferred_element_type=jnp.float32)
            ysend_ref[0, rows(c)] = p1.astype(jnp.bfloat16)
            y_rdma(0, c).start()

        for l in range(N_LAYERS):
            for c in range(NC):
                y_rdma(l, c).wait_recv()
                hsum = (ysend_ref[l, rows(c)].astype(jnp.float32)
                        + ybuf_ref[l, rows(c)].astype(jnp.float32))
                hb = jnp.maximum(hsum, 0.0).astype(jnp.bfloat16)
                p2 = jnp.dot(hb, woutb_ref[l],
                             preferred_element_type=jnp.float32)
                xsend_ref[l, rows(c)] = p2.astype(jnp.bfloat16)
                x_rdma(l, c).start()
            for c in range(NC):
                x_rdma(l, c).wait_recv()
                asum = (xsend_ref[l, rows(c)].astype(jnp.float32)
                        + xbuf_ref[l, rows(c)].astype(jnp.float32))
                if l == N_LAYERS - 1:
                    out_ref[rows(c), :] = asum
                else:
                    p1 = jnp.dot(asum.astype(jnp.bfloat16), winb_ref[l + 1],
                                 preferred_element_type=jnp.float32)
                    ysend_ref[l + 1, rows(c)] = p1.astype(jnp.bfloat16)
                    y_rdma(l + 1, c).start()

        for l in range(N_LAYERS):
            for c in range(NC):
                y_rdma(l, c).wait_send()
                x_rdma(l, c).wait_send()

    return pl.pallas_call(
        body,
        out_shape=jax.ShapeDtypeStruct((b, d_loc), jnp.float32),
        in_specs=[pl.BlockSpec(memory_space=pltpu.VMEM)] * 7,
        out_specs=pl.BlockSpec(memory_space=pltpu.VMEM),
        scratch_shapes=[
            pltpu.VMEM((N_LAYERS, d_in, h_loc), jnp.bfloat16),
            pltpu.VMEM((N_LAYERS, h_loc, d_loc), jnp.bfloat16),
            pltpu.VMEM((N_LAYERS, b, h_loc), jnp.bfloat16),
            pltpu.VMEM((N_LAYERS, b, d_loc), jnp.bfloat16),
            pltpu.VMEM((N_LAYERS, b, h_loc), jnp.bfloat16),
            pltpu.VMEM((N_LAYERS, b, d_loc), jnp.bfloat16),
            pltpu.SemaphoreType.DMA((N_LAYERS, NC)),
            pltpu.SemaphoreType.DMA((N_LAYERS, NC)),
            pltpu.SemaphoreType.DMA((N_LAYERS, NC)),
            pltpu.SemaphoreType.DMA((N_LAYERS, NC)),
        ],
        compiler_params=pltpu.CompilerParams(collective_id=0),
    )(x, Win0, Wout0, Win1, Wout1, Win2, Wout2)


# device time: 28251 ns/iter; 1.2544x vs baseline; 1.0683x over previous
import jax
import jax.numpy as jnp
from jax import lax
from jax.experimental import pallas as pl
from jax.experimental.pallas import tpu as pltpu

N_LAYERS = 3
NC = 4


def kernel(x, Win0, Wout0, Win1, Wout1, Win2, Wout2):
    b, d_loc = x.shape
    d_in, h_loc = Win0.shape
    bc = b // NC

    def rows(c):
        return pl.ds(c * bc, bc)

    def body(x_ref, win0_ref, wout0_ref, win1_ref, wout1_ref, win2_ref,
             wout2_ref, out_ref, winb_ref, woutb_ref, ysend_ref, xsend_ref,
             ybuf_ref, xbuf_ref, ysend_sem, yrecv_sem, xsend_sem, xrecv_sem):
        my_x = lax.axis_index("x")
        my_y = lax.axis_index("y")
        y_peer = (my_x, 1 - my_y)
        x_peer = (1 - my_x, my_y)

        barrier_sem = pltpu.get_barrier_semaphore()
        for nbr in (y_peer, x_peer):
            pl.semaphore_signal(
                barrier_sem, inc=1,
                device_id=nbr, device_id_type=pl.DeviceIdType.MESH,
            )

        wins = (win0_ref, win1_ref, win2_ref)
        wouts = (wout0_ref, wout1_ref, wout2_ref)
        for l in range(N_LAYERS):
            winb_ref[l, :, :] = wins[l][:, :].astype(jnp.bfloat16)
            woutb_ref[l, :, :] = wouts[l][:, :].astype(jnp.bfloat16)

        pl.semaphore_wait(barrier_sem, 2)

        def y_rdma(l, c):
            return pltpu.make_async_remote_copy(
                src_ref=ysend_ref.at[l, rows(c)],
                dst_ref=ybuf_ref.at[l, rows(c)],
                send_sem=ysend_sem.at[l, c],
                recv_sem=yrecv_sem.at[l, c],
                device_id=y_peer,
                device_id_type=pl.DeviceIdType.MESH,
            )

        def x_rdma(l, c):
            return pltpu.make_async_remote_copy(
                src_ref=xsend_ref.at[l, rows(c)],
                dst_ref=xbuf_ref.at[l, rows(c)],
                send_sem=xsend_sem.at[l, c],
                recv_sem=xrecv_sem.at[l, c],
                device_id=x_peer,
                device_id_type=pl.DeviceIdType.MESH,
            )

        act = x_ref[:, :].astype(jnp.bfloat16)
        for c in range(NC):
            p1 = jnp.dot(act[c * bc:(c + 1) * bc, :], winb_ref[0],
                         preferred_element_type=jnp.float32)
            ysend_ref[0, rows(c)] = p1.astype(jnp.bfloat16)
            y_rdma(0, c).start()

        for l in range(N_LAYERS):
            for c in range(NC):
                y_rdma(l, c).wait_recv()
                hsum = (ysend_ref[l, rows(c)].astype(jnp.float32)
                        + ybuf_ref[l, rows(c)].astype(jnp.float32))
                hb = jnp.maximum(hsum, 0.0).astype(jnp.bfloat16)
                p2 = jnp.dot(hb, woutb_ref[l],
                             preferred_element_type=jnp.float32)
                xsend_ref[l, rows(c)] = p2.astype(jnp.bfloat16)
                x_rdma(l, c).start()
            for c in range(NC):
                x_rdma(l, c).wait_recv()
                asum = (xsend_ref[l, rows(c)].astype(jnp.float32)
                        + xbuf_ref[l, rows(c)].astype(jnp.float32))
                if l == N_LAYERS - 1:
                    out_ref[rows(c), :] = asum
                else:
                    p1 = jnp.dot(asum.astype(jnp.bfloat16), winb_ref[l + 1],
                                 preferred_element_type=jnp.float32)
                    ysend_ref[l + 1, rows(c)] = p1.astype(jnp.bfloat16)
                    y_rdma(l + 1, c).start()

        for l in range(N_LAYERS):
            for c in range(NC):
                y_rdma(l, c).wait_send()
                x_rdma(l, c).wait_send()

    return pl.pallas_call(
        body,
        out_shape=jax.ShapeDtypeStruct((b, d_loc), jnp.float32),
        in_specs=[pl.BlockSpec(memory_space=pltpu.VMEM)] * 7,
        out_specs=pl.BlockSpec(memory_space=pltpu.VMEM),
        scratch_shapes=[
            pltpu.VMEM((N_LAYERS, d_in, h_loc), jnp.bfloat16),
            pltpu.VMEM((N_LAYERS, h_loc, d_loc), jnp.bfloat16),
            pltpu.VMEM((N_LAYERS, b, h_loc), jnp.bfloat16),
            pltpu.VMEM((N_LAYERS, b, d_loc), jnp.bfloat16),
            pltpu.VMEM((N_LAYERS, b, h_loc), jnp.bfloat16),
            pltpu.VMEM((N_LAYERS, b, d_loc), jnp.bfloat16),
            pltpu.SemaphoreType.DMA((N_LAYERS, NC)),
            pltpu.SemaphoreType.DMA((N_LAYERS, NC)),
            pltpu.SemaphoreType.DMA((N_LAYERS, NC)),
            pltpu.SemaphoreType.DMA((N_LAYERS, NC)),
        ],
        compiler_params=pltpu.CompilerParams(collective_id=0),
    )(x, Win0, Wout0, Win1, Wout1, Win2, Wout2)
